# Initial kernel scaffold; baseline (speedup 1.0000x reference)
import functools

import jax
import jax.numpy as jnp
from jax import lax
from jax.experimental import pallas as pl
from jax.experimental.pallas import tpu as pltpu
from jax.experimental.pallas import tpu_sc as plsc

N = 10000
D = 128
NC = 2
NS = 16
NW = NC * NS
GB = 128
ROWS_PER_TILE = N // NS

_mesh = plsc.VectorSubcoreMesh(core_axis_name="c", subcore_axis_name="s")


def _make_spmm(groups_per_tile: int):
    @functools.partial(
        pl.kernel,
        mesh=_mesh,
        out_type=jax.ShapeDtypeStruct((NC, N, D), jnp.float32),
        scratch_types=[
            pltpu.VMEM((GB,), jnp.int32),
            pltpu.VMEM((GB,), jnp.int32),
            pltpu.VMEM((GB,), jnp.float32),
            pltpu.VMEM((GB, D), jnp.float32),
            pltpu.VMEM_SHARED((N, D), jnp.float32),
            pltpu.SemaphoreType.DMA,
        ],
    )
    def spmm(row_hbm, col_hbm, val_hbm, prev_hbm, zeros_hbm, out_hbm,
             rowb, colb, valb, rbuf, acc, sem):
        cid = lax.axis_index("c")
        sid = lax.axis_index("s")
        wid = sid * NC + cid
        r0 = sid * ROWS_PER_TILE

        pltpu.sync_copy(zeros_hbm.at[pl.ds(r0, ROWS_PER_TILE)],
                        acc.at[pl.ds(r0, ROWS_PER_TILE)])
        plsc.subcore_barrier()

        base = wid * (groups_per_tile * GB)

        def group_body(g, carry):
            off = base + g * GB
            pltpu.sync_copy(row_hbm.at[pl.ds(off, GB)], rowb)
            pltpu.sync_copy(col_hbm.at[pl.ds(off, GB)], colb)
            pltpu.sync_copy(val_hbm.at[pl.ds(off, GB)], valb)
            pltpu.async_copy(prev_hbm.at[colb], rbuf, sem).wait()

            def scale_body(e, c2):
                sv = jnp.full((16,), valb[e], jnp.float32)
                for j in range(D // 16):
                    rbuf[e, pl.ds(j * 16, 16)] = rbuf[e, pl.ds(j * 16, 16)] * sv
                return c2

            lax.fori_loop(0, GB, scale_body, 0, unroll=False)
            pltpu.sync_copy(rbuf, acc.at[rowb], add=True)
            return carry

        lax.fori_loop(0, groups_per_tile, group_body, 0, unroll=False)
        plsc.subcore_barrier()

        pltpu.sync_copy(acc.at[pl.ds(r0, ROWS_PER_TILE)],
                        out_hbm.at[cid, pl.ds(r0, ROWS_PER_TILE)])

    return spmm


_ROW_BLK = 1000


def _dense_body(a0_ref, a1_ref, prev_ref, w1_ref, w2_ref, b1_ref, b2_ref,
                out_ref):
    a = a0_ref[...] + a1_ref[...]
    x2 = prev_ref[...] * a
    out_ref[...] = (
        jnp.dot(a, w1_ref[...], preferred_element_type=jnp.float32)
        + jnp.dot(x2, w2_ref[...], preferred_element_type=jnp.float32)
        + b1_ref[...] + b2_ref[...]
    )


def _dense(a0, a1, prev, w1, w2, b1, b2):
    grid = (N // _ROW_BLK,)
    row_spec = pl.BlockSpec((_ROW_BLK, D), lambda i: (i, 0))
    full_spec = pl.BlockSpec((D, D), lambda i: (0, 0))
    bias_spec = pl.BlockSpec((1, D), lambda i: (0, 0))
    return pl.pallas_call(
        _dense_body,
        grid=grid,
        in_specs=[row_spec, row_spec, row_spec, full_spec, full_spec,
                  bias_spec, bias_spec],
        out_specs=row_spec,
        out_shape=jax.ShapeDtypeStruct((N, D), jnp.float32),
    )(a0, a1, prev, w1, w2, b1, b2)


def kernel(L_I_indices, L_I_values, prev_embeddings, W1, W2, b1, b2):
    e = L_I_values.shape[0]
    groups_per_tile = -(-e // (NW * GB))
    e_pad = groups_per_tile * NW * GB
    pad = e_pad - e

    row = L_I_indices[0]
    col = L_I_indices[1]
    if pad:
        zi = jnp.zeros((pad,), jnp.int32)
        row = jnp.concatenate([row, zi])
        col = jnp.concatenate([col, zi])
        vals = jnp.concatenate([L_I_values, jnp.zeros((pad,), jnp.float32)])
    else:
        vals = L_I_values

    zeros = jnp.zeros((N, D), jnp.float32)
    partial = _make_spmm(groups_per_tile)(row, col, vals, prev_embeddings,
                                          zeros)
    return _dense(partial[0], partial[1], prev_embeddings, W1, W2, b1, b2)


# speedup vs baseline: 3.8166x; 3.8166x over previous
import functools

import jax
import jax.numpy as jnp
from jax import lax
from jax.experimental import pallas as pl
from jax.experimental.pallas import tpu as pltpu
from jax.experimental.pallas import tpu_sc as plsc

N = 10000
D = 128
NC = 2
NS = 16
NW = NC * NS
GB = 128
NP = 10240
ROWS_PER_TILE = NP // NS

_mesh = plsc.VectorSubcoreMesh(core_axis_name="c", subcore_axis_name="s")


def _make_spmm(groups_per_tile: int):
    @functools.partial(
        pl.kernel,
        mesh=_mesh,
        out_type=jax.ShapeDtypeStruct((NC, NP, D), jnp.float32),
        scratch_types=[
            pltpu.VMEM((GB,), jnp.int32),
            pltpu.VMEM((GB,), jnp.int32),
            pltpu.VMEM((GB,), jnp.float32),
            pltpu.VMEM((GB, D), jnp.float32),
            pltpu.VMEM_SHARED((NP, D), jnp.float32),
            pltpu.SemaphoreType.DMA,
        ],
    )
    def spmm(row_hbm, col_hbm, val_hbm, prev_hbm, zeros_hbm, out_hbm,
             rowb, colb, valb, rbuf, acc, sem):
        cid = lax.axis_index("c")
        sid = lax.axis_index("s")
        wid = sid * NC + cid
        r0 = sid * ROWS_PER_TILE

        pltpu.sync_copy(zeros_hbm.at[pl.ds(r0, ROWS_PER_TILE)],
                        acc.at[pl.ds(r0, ROWS_PER_TILE)])
        plsc.subcore_barrier()

        base = wid * (groups_per_tile * GB)

        def group_body(g, carry):
            off = base + g * GB
            pltpu.sync_copy(row_hbm.at[pl.ds(off, GB)], rowb)
            pltpu.sync_copy(col_hbm.at[pl.ds(off, GB)], colb)
            pltpu.sync_copy(val_hbm.at[pl.ds(off, GB)], valb)
            pltpu.async_copy(prev_hbm.at[colb], rbuf, sem).wait()

            def scale_body(k, c2):
                vv = valb[pl.ds(k * 16, 16)]
                for l in range(16):
                    sv = jnp.full((16,), vv[l], jnp.float32)
                    e = k * 16 + l
                    for j in range(D // 16):
                        rbuf[e, pl.ds(j * 16, 16)] = (
                            rbuf[e, pl.ds(j * 16, 16)] * sv)
                return c2

            lax.fori_loop(0, GB // 16, scale_body, 0, unroll=False)
            pltpu.sync_copy(rbuf, acc.at[rowb], add=True)
            return carry

        lax.fori_loop(0, groups_per_tile, group_body, 0, unroll=False)
        plsc.subcore_barrier()

        pltpu.sync_copy(acc.at[pl.ds(r0, ROWS_PER_TILE)],
                        out_hbm.at[cid, pl.ds(r0, ROWS_PER_TILE)])

    return spmm


_ROW_BLK = 1000


def _dense_body(a0_ref, a1_ref, prev_ref, w1_ref, w2_ref, b1_ref, b2_ref,
                out_ref):
    a = a0_ref[...] + a1_ref[...]
    x2 = prev_ref[...] * a
    out_ref[...] = (
        jnp.dot(a, w1_ref[...], preferred_element_type=jnp.float32)
        + jnp.dot(x2, w2_ref[...], preferred_element_type=jnp.float32)
        + b1_ref[...] + b2_ref[...]
    )


def _dense(a0, a1, prev, w1, w2, b1, b2):
    grid = (N // _ROW_BLK,)
    row_spec = pl.BlockSpec((_ROW_BLK, D), lambda i: (i, 0))
    full_spec = pl.BlockSpec((D, D), lambda i: (0, 0))
    bias_spec = pl.BlockSpec((1, D), lambda i: (0, 0))
    return pl.pallas_call(
        _dense_body,
        grid=grid,
        in_specs=[row_spec, row_spec, row_spec, full_spec, full_spec,
                  bias_spec, bias_spec],
        out_specs=row_spec,
        out_shape=jax.ShapeDtypeStruct((N, D), jnp.float32),
    )(a0, a1, prev, w1, w2, b1, b2)


def kernel(L_I_indices, L_I_values, prev_embeddings, W1, W2, b1, b2):
    e = L_I_values.shape[0]
    groups_per_tile = -(-e // (NW * GB))
    e_pad = groups_per_tile * NW * GB
    pad = e_pad - e

    row = L_I_indices[0]
    col = L_I_indices[1]
    if pad:
        zi = jnp.zeros((pad,), jnp.int32)
        row = jnp.concatenate([row, zi])
        col = jnp.concatenate([col, zi])
        vals = jnp.concatenate([L_I_values, jnp.zeros((pad,), jnp.float32)])
    else:
        vals = L_I_values

    zeros = jnp.zeros((NP, D), jnp.float32)
    partial = _make_spmm(groups_per_tile)(row, col, vals, prev_embeddings,
                                          zeros)
    return _dense(partial[0, :N], partial[1, :N], prev_embeddings, W1, W2,
                  b1, b2)
